# baseline (speedup 1.0000x reference)
import functools

import jax
import jax.numpy as jnp
from jax import lax
from jax.experimental import pallas as pl
from jax.experimental.pallas import tpu as pltpu
from jax.experimental.pallas import tpu_sc as plsc

K = 8192
D = 256
NT = 16384
BM = 512
N_BLOCKS = NT // BM

_GROUPS = ((0, 2736), (2736, 5472), (5472, K))


def _wn_body(w_ref, wn_ref):
    w = w_ref[...]
    ones = jnp.ones((8, D), jnp.float32)
    wn8 = lax.dot_general(ones, w * w, (((1,), (1,)), ((), ())),
                          preferred_element_type=jnp.float32)
    wn_ref[...] = wn8[:1, :]


def _codebook_norms(w):
    return pl.pallas_call(
        _wn_body,
        out_shape=jax.ShapeDtypeStruct((1, K), jnp.float32),
    )(w)


def _aligned(lo, hi):
    return (lo // 128) * 128, ((hi + 127) // 128) * 128


def _distance_argmin_body(z_ref, w_ref, wn_ref, pen_ref, col_ref,
                          idx_ref, loss_ref, acc_ref):
    i = pl.program_id(0)

    @pl.when(i == 0)
    def _():
        acc_ref[0, 0] = 0.0

    zb = z_ref[...]
    mm2 = lax.dot_general(zb + zb, w_ref[...], (((1,), (1,)), ((), ())),
                          preferred_element_type=jnp.float32)
    zn = jnp.sum(zb * zb, axis=1, keepdims=True)
    d = (zn + wn_ref[...]) - mm2

    acc_v = acc_i = None
    for g, (lo, hi) in enumerate(_GROUPS):
        lo_i = ((lo + 127) // 128) * 128
        hi_i = (hi // 128) * 128
        pieces = [(d[:, lo_i:hi_i], col_ref[:, lo_i:hi_i])]
        if lo != lo_i:
            la = lo_i - 128
            pieces.append((d[:, la:lo_i] + pen_ref[g:g + 1, la:lo_i],
                           col_ref[:, la:lo_i]))
        if hi != hi_i:
            ha = hi_i + 128
            pieces.append((d[:, hi_i:ha] + pen_ref[g:g + 1, hi_i:ha],
                           col_ref[:, hi_i:ha]))
        w_v = pieces[0][0].min(axis=1, keepdims=True)
        for dm, _ in pieces[1:]:
            w_v = jnp.minimum(w_v, dm.min(axis=1, keepdims=True))
        w_i = None
        for dm, cols in pieces:
            pi = jnp.min(jnp.where(dm == w_v, cols, float(K)),
                         axis=1, keepdims=True)
            w_i = pi if w_i is None else jnp.minimum(w_i, pi)
        if acc_v is None:
            acc_v, acc_i = w_v, w_i
        else:
            av = acc_v.astype(jnp.bfloat16).astype(jnp.float32)
            take = w_v < av
            acc_v = jnp.where(take, w_v, acc_v)
            acc_i = jnp.where(take, w_i, acc_i)

    idx_ref[...] = acc_i.astype(jnp.int32).reshape((BM,))
    acc_ref[0, 0] += jnp.sum(acc_v)

    @pl.when(i == N_BLOCKS - 1)
    def _():
        loss_ref[0, 0] = acc_ref[0, 0] * (1.25 / (NT * D))


def _distance_argmin(z_flat, w, wn, pen, colf):
    return pl.pallas_call(
        _distance_argmin_body,
        grid=(N_BLOCKS,),
        in_specs=[
            pl.BlockSpec((BM, D), lambda i: (i, 0)),
            pl.BlockSpec((K, D), lambda i: (0, 0)),
            pl.BlockSpec((1, K), lambda i: (0, 0)),
            pl.BlockSpec((len(_GROUPS), K), lambda i: (0, 0)),
            pl.BlockSpec((1, K), lambda i: (0, 0)),
        ],
        out_specs=[
            pl.BlockSpec((BM,), lambda i: (i,)),
            pl.BlockSpec((1, 1), lambda i: (0, 0),
                         memory_space=pltpu.SMEM),
        ],
        out_shape=[
            jax.ShapeDtypeStruct((NT,), jnp.int32),
            jax.ShapeDtypeStruct((1, 1), jnp.float32),
        ],
        scratch_shapes=[
            pltpu.SMEM((1, 1), jnp.float32),
        ],
        compiler_params=pltpu.CompilerParams(
            dimension_semantics=("arbitrary",)),
    )(z_flat, w, wn, pen, colf)


_SC_INFO = plsc.get_sparse_core_info()
_NC = _SC_INFO.num_cores
_NS = _SC_INFO.num_subcores
_NW = _NC * _NS
_ROWS_PER_W = NT // _NW
_CHUNK = 128


_NCHUNK = _ROWS_PER_W // _CHUNK


@functools.partial(
    pl.kernel,
    out_type=jax.ShapeDtypeStruct((NT, D), jnp.float32),
    mesh=plsc.VectorSubcoreMesh(core_axis_name="c", subcore_axis_name="s"),
    scratch_types=[
        pltpu.VMEM((_NCHUNK, _CHUNK), jnp.int32),
        pltpu.VMEM((_CHUNK, D), jnp.float32),
        pltpu.VMEM((_CHUNK, D), jnp.float32),
        pltpu.SemaphoreType.DMA,
        pltpu.SemaphoreType.DMA,
    ],
)
def _sc_gather(w_hbm, idx_hbm, out_hbm, idx_v, rows0, rows1, sem0, sem1):
    wid = lax.axis_index("s") * _NC + lax.axis_index("c")
    base = wid * _ROWS_PER_W
    pltpu.sync_copy(idx_hbm.at[pl.ds(wid * _NCHUNK, _NCHUNK)], idx_v)
    bufs = (rows0, rows1)
    sems = (sem0, sem1)
    cps = [None, None]
    cps[0] = pltpu.async_copy(w_hbm.at[idx_v.at[0]], rows0, sem0)
    cps[1] = pltpu.async_copy(w_hbm.at[idx_v.at[1]], rows1, sem1)
    for c in range(_NCHUNK):
        b = c % 2
        cps[b].wait()
        pltpu.sync_copy(bufs[b], out_hbm.at[pl.ds(base + c * _CHUNK, _CHUNK)])
        if c + 2 < _NCHUNK:
            cps[b] = pltpu.async_copy(w_hbm.at[idx_v.at[c + 2]],
                                      bufs[b], sems[b])


_ST_BM = 2048


def _straight_through_body(z_ref, q_ref, o_ref):
    z = z_ref[...]
    o_ref[...] = z + (q_ref[...] - z)


def _straight_through(z_flat, z_q):
    return pl.pallas_call(
        _straight_through_body,
        grid=(NT // _ST_BM,),
        in_specs=[
            pl.BlockSpec((_ST_BM, D), lambda i: (i, 0)),
            pl.BlockSpec((_ST_BM, D), lambda i: (i, 0)),
        ],
        out_specs=pl.BlockSpec((_ST_BM, D), lambda i: (i, 0)),
        out_shape=jax.ShapeDtypeStruct((NT, D), jnp.float32),
    )(z_flat, z_q)


def kernel(z, W):
    z_flat = z.reshape(-1, D)
    wn = _codebook_norms(W)
    cols = lax.broadcasted_iota(jnp.int32, (1, K), 1)
    pen = jnp.stack([
        jnp.where((cols >= lo) & (cols < hi), 0.0, jnp.inf).reshape(K)
        for lo, hi in _GROUPS])
    colf = cols.astype(jnp.float32)
    indices, loss2d = _distance_argmin(z_flat, W, wn, pen, colf)
    z_q = _sc_gather(W, indices.reshape(NT // _CHUNK, _CHUNK))
    z_q_st = _straight_through(z_flat, z_q)
    return (z_q_st.reshape(z.shape), loss2d.reshape(()), indices)


# speedup vs baseline: 1.0294x; 1.0294x over previous
import functools

import jax
import jax.numpy as jnp
from jax import lax
from jax.experimental import pallas as pl
from jax.experimental.pallas import tpu as pltpu
from jax.experimental.pallas import tpu_sc as plsc

K = 8192
D = 256
NT = 16384
BM = 512
N_BLOCKS = NT // BM

_GROUPS = ((0, 2736), (2736, 5472), (5472, K))


def _wn_body(w_ref, wn_ref):
    w = w_ref[...]
    ones = jnp.ones((8, D), jnp.float32)
    wn8 = lax.dot_general(ones, w * w, (((1,), (1,)), ((), ())),
                          preferred_element_type=jnp.float32)
    wn_ref[...] = wn8[:1, :]


def _codebook_norms(w):
    return pl.pallas_call(
        _wn_body,
        out_shape=jax.ShapeDtypeStruct((1, K), jnp.float32),
    )(w)


def _aligned(lo, hi):
    return (lo // 128) * 128, ((hi + 127) // 128) * 128


def _distance_argmin_body(z_ref, w_ref, wn_ref, pen_ref, col_ref,
                          idx_ref, loss_ref, acc_ref):
    i = pl.program_id(0)

    @pl.when(i == 0)
    def _():
        acc_ref[0, 0] = 0.0

    zb = z_ref[...]
    mm2 = lax.dot_general(zb + zb, w_ref[...], (((1,), (1,)), ((), ())),
                          preferred_element_type=jnp.float32)
    zn = jnp.sum(zb * zb, axis=1, keepdims=True)
    d = (zn + wn_ref[...]) - mm2

    acc_v = acc_i = None
    for g, (lo, hi) in enumerate(_GROUPS):
        lo_a, hi_a = _aligned(lo, hi)
        dm = d[:, lo_a:hi_a] + pen_ref[g:g + 1, lo_a:hi_a]
        w_v = jnp.min(dm, axis=1, keepdims=True)
        w_i = jnp.min(jnp.where(dm == w_v, col_ref[:, lo_a:hi_a], float(K)),
                      axis=1, keepdims=True)
        if acc_v is None:
            acc_v, acc_i = w_v, w_i
        else:
            av = acc_v.astype(jnp.bfloat16).astype(jnp.float32)
            take = w_v < av
            acc_v = jnp.where(take, w_v, acc_v)
            acc_i = jnp.where(take, w_i, acc_i)

    idx_ref[...] = acc_i.astype(jnp.int32).reshape((BM,))
    acc_ref[0, 0] += jnp.sum(acc_v)

    @pl.when(i == N_BLOCKS - 1)
    def _():
        loss_ref[0, 0] = acc_ref[0, 0] * (1.25 / (NT * D))


def _distance_argmin(z_flat, w, wn, pen, colf):
    return pl.pallas_call(
        _distance_argmin_body,
        grid=(N_BLOCKS,),
        in_specs=[
            pl.BlockSpec((BM, D), lambda i: (i, 0)),
            pl.BlockSpec((K, D), lambda i: (0, 0)),
            pl.BlockSpec((1, K), lambda i: (0, 0)),
            pl.BlockSpec((len(_GROUPS), K), lambda i: (0, 0)),
            pl.BlockSpec((1, K), lambda i: (0, 0)),
        ],
        out_specs=[
            pl.BlockSpec((BM,), lambda i: (i,)),
            pl.BlockSpec((1, 1), lambda i: (0, 0),
                         memory_space=pltpu.SMEM),
        ],
        out_shape=[
            jax.ShapeDtypeStruct((NT,), jnp.int32),
            jax.ShapeDtypeStruct((1, 1), jnp.float32),
        ],
        scratch_shapes=[
            pltpu.SMEM((1, 1), jnp.float32),
        ],
        compiler_params=pltpu.CompilerParams(
            dimension_semantics=("arbitrary",)),
    )(z_flat, w, wn, pen, colf)


_SC_INFO = plsc.get_sparse_core_info()
_NC = _SC_INFO.num_cores
_NS = _SC_INFO.num_subcores
_NW = _NC * _NS
_ROWS_PER_W = NT // _NW
_CHUNK = 128


_NCHUNK = _ROWS_PER_W // _CHUNK


@functools.partial(
    pl.kernel,
    out_type=jax.ShapeDtypeStruct((NT, D), jnp.float32),
    mesh=plsc.VectorSubcoreMesh(core_axis_name="c", subcore_axis_name="s"),
    scratch_types=[
        pltpu.VMEM((_NCHUNK, _CHUNK), jnp.int32),
        pltpu.VMEM((_CHUNK, D), jnp.float32),
        pltpu.VMEM((_CHUNK, D), jnp.float32),
        pltpu.SemaphoreType.DMA,
        pltpu.SemaphoreType.DMA,
    ],
)
def _sc_gather(w_hbm, idx_hbm, out_hbm, idx_v, rows0, rows1, sem0, sem1):
    wid = lax.axis_index("s") * _NC + lax.axis_index("c")
    base = wid * _ROWS_PER_W
    pltpu.sync_copy(idx_hbm.at[pl.ds(wid * _NCHUNK, _NCHUNK)], idx_v)
    bufs = (rows0, rows1)
    sems = (sem0, sem1)
    cps = [None, None]
    cps[0] = pltpu.async_copy(w_hbm.at[idx_v.at[0]], rows0, sem0)
    cps[1] = pltpu.async_copy(w_hbm.at[idx_v.at[1]], rows1, sem1)
    for c in range(_NCHUNK):
        b = c % 2
        cps[b].wait()
        pltpu.sync_copy(bufs[b], out_hbm.at[pl.ds(base + c * _CHUNK, _CHUNK)])
        if c + 2 < _NCHUNK:
            cps[b] = pltpu.async_copy(w_hbm.at[idx_v.at[c + 2]],
                                      bufs[b], sems[b])


_ST_BM = 2048


def _straight_through_body(z_ref, q_ref, o_ref):
    z = z_ref[...]
    o_ref[...] = z + (q_ref[...] - z)


def _straight_through(z_flat, z_q):
    return pl.pallas_call(
        _straight_through_body,
        grid=(NT // _ST_BM,),
        in_specs=[
            pl.BlockSpec((_ST_BM, D), lambda i: (i, 0)),
            pl.BlockSpec((_ST_BM, D), lambda i: (i, 0)),
        ],
        out_specs=pl.BlockSpec((_ST_BM, D), lambda i: (i, 0)),
        out_shape=jax.ShapeDtypeStruct((NT, D), jnp.float32),
    )(z_flat, z_q)


def kernel(z, W):
    z_flat = z.reshape(-1, D)
    wn = _codebook_norms(W)
    cols = lax.broadcasted_iota(jnp.int32, (1, K), 1)
    pen = jnp.stack([
        jnp.where((cols >= lo) & (cols < hi), 0.0, jnp.inf).reshape(K)
        for lo, hi in _GROUPS])
    colf = cols.astype(jnp.float32)
    indices, loss2d = _distance_argmin(z_flat, W, wn, pen, colf)
    z_q = _sc_gather(W, indices.reshape(NT // _CHUNK, _CHUNK))
    z_q_st = _straight_through(z_flat, z_q)
    return (z_q_st.reshape(z.shape), loss2d.reshape(()), indices)
